# baseline (speedup 1.0000x reference)
import functools

import jax
import jax.numpy as jnp
from jax.experimental import pallas as pl
from jax.experimental.pallas import tpu as pltpu

N = 4096
D_OUT = 8


def _proj_kernel(x_ref, w1_ref, b1_ref, w2_ref, asrc_ref, adst_ref,
                 g_ref, a_s_ref, a_d_ref):
    x = x_ref[...]
    h = jax.lax.dot_general(x, w1_ref[...], (((1,), (1,)), ((), ())),
                            preferred_element_type=jnp.float32)
    h = h + b1_ref[...]
    h = jnp.where(h >= 0, h, 0.01 * h)
    g = jax.lax.dot_general(h, w2_ref[...], (((1,), (1,)), ((), ())),
                            preferred_element_type=jnp.float32)
    g_ref[...] = g
    a_s_ref[...] = jax.lax.dot_general(g, asrc_ref[...], (((1,), (0,)), ((), ())),
                                       preferred_element_type=jnp.float32)
    a_d_ref[...] = jax.lax.dot_general(g, adst_ref[...], (((1,), (0,)), ((), ())),
                                       preferred_element_type=jnp.float32)


def _attn_kernel(adj_ref, g_ref, a_s_ref, a_d_ref, bias_ref, out_ref,
                 m_ref, s_ref, acc_ref, *, ib, jb, ni):
    j = pl.program_id(0)
    i = pl.program_id(1)

    @pl.when(i == 0)
    def _init():
        m_ref[...] = jnp.full_like(m_ref, -3.4e38)
        s_ref[...] = jnp.zeros_like(s_ref)
        acc_ref[...] = jnp.zeros_like(acc_ref)

    a = adj_ref[...]
    rows = i * ib + jax.lax.broadcasted_iota(jnp.int32, (ib, jb), 0)
    cols = j * jb + jax.lax.broadcasted_iota(jnp.int32, (ib, jb), 1)
    mask = (a != 0) | (rows == cols)

    z = a_s_ref[...] + a_d_ref[...]
    e = jnp.where(z >= 0, z, 0.2 * z)

    e_m = jnp.where(mask, e, -3.4e38)
    bm = jnp.max(e_m, axis=0, keepdims=True)
    m_new = jnp.maximum(m_ref[...], bm)
    corr = jnp.exp(m_ref[...] - m_new)
    p = jnp.where(mask, jnp.exp(e - m_new), 0.0)

    m_ref[...] = m_new
    s_ref[...] = s_ref[...] * corr + jnp.sum(p, axis=0, keepdims=True)
    acc_ref[...] = acc_ref[...] * corr + jax.lax.dot_general(
        g_ref[...], p, (((0,), (0,)), ((), ())),
        preferred_element_type=jnp.float32)

    @pl.when(i == ni - 1)
    def _fini():
        out_ref[...] = acc_ref[...] / (s_ref[...] + 1e-16) + bias_ref[...]


@jax.jit
def kernel(x, adj, W1, b1, W2, att_src, att_dst, bias):
    n = x.shape[0]

    g, a_s, a_d = pl.pallas_call(
        _proj_kernel,
        out_shape=(
            jax.ShapeDtypeStruct((n, D_OUT), jnp.float32),
            jax.ShapeDtypeStruct((n, 1), jnp.float32),
            jax.ShapeDtypeStruct((n, 1), jnp.float32),
        ),
    )(x, W1, b1.reshape(1, -1), W2,
      att_src.reshape(-1, 1), att_dst.reshape(-1, 1))

    ib, jb = 1024, 512
    ni = n // ib
    nj = n // jb

    out_t = pl.pallas_call(
        functools.partial(_attn_kernel, ib=ib, jb=jb, ni=ni),
        grid=(nj, ni),
        in_specs=[
            pl.BlockSpec((ib, jb), lambda j, i: (i, j)),
            pl.BlockSpec((ib, D_OUT), lambda j, i: (i, 0)),
            pl.BlockSpec((ib, 1), lambda j, i: (i, 0)),
            pl.BlockSpec((1, jb), lambda j, i: (0, j)),
            pl.BlockSpec((D_OUT, 1), lambda j, i: (0, 0)),
        ],
        out_specs=pl.BlockSpec((D_OUT, jb), lambda j, i: (0, j)),
        out_shape=jax.ShapeDtypeStruct((D_OUT, n), jnp.float32),
        scratch_shapes=[
            pltpu.VMEM((1, jb), jnp.float32),
            pltpu.VMEM((1, jb), jnp.float32),
            pltpu.VMEM((D_OUT, jb), jnp.float32),
        ],
        compiler_params=pltpu.CompilerParams(
            dimension_semantics=("parallel", "arbitrary")),
    )(adj, g, a_s, a_d.reshape(1, -1), bias.reshape(-1, 1))

    return out_t.T


# speedup vs baseline: 1.1368x; 1.1368x over previous
import functools

import jax
import jax.numpy as jnp
from jax.experimental import pallas as pl
from jax.experimental.pallas import tpu as pltpu

D_OUT = 8
LOG2E = 1.4426950408889634


def _proj_kernel(x_ref, w1_ref, b1_ref, w2_ref, asrc_ref, adst_ref,
                 g_ref, a_s_ref, a_d_ref, m_ref):
    x = x_ref[...]
    h = jax.lax.dot_general(x, w1_ref[...], (((1,), (1,)), ((), ())),
                            preferred_element_type=jnp.float32)
    h = h + b1_ref[...]
    h = jnp.where(h >= 0, h, 0.01 * h)
    g = jax.lax.dot_general(h, w2_ref[...], (((1,), (1,)), ((), ())),
                            preferred_element_type=jnp.float32)
    g_ref[:, :D_OUT] = g.astype(jnp.bfloat16)
    g_ref[:, D_OUT:] = jnp.ones_like(g_ref[:, D_OUT:])
    a_s = LOG2E * jax.lax.dot_general(g, asrc_ref[...], (((1,), (0,)), ((), ())),
                                      preferred_element_type=jnp.float32)
    a_d = LOG2E * jax.lax.dot_general(g, adst_ref[...], (((1,), (0,)), ((), ())),
                                      preferred_element_type=jnp.float32)
    a_s_ref[...] = a_s
    a_d_ref[...] = a_d
    t = jnp.max(a_s) + a_d
    m_ref[...] = jnp.maximum(t, 0.2 * t)


def _attn_kernel(adj_ref, g_ref, a_s_ref, a_d_ref, m_ref, bias_ref, out_ref,
                 acc_ref, *, ib, jb, ni):
    j = pl.program_id(0)
    i = pl.program_id(1)

    @pl.when(i == 0)
    def _init():
        acc_ref[...] = jnp.zeros_like(acc_ref)

    a = adj_ref[...]
    z = a_s_ref[...] + a_d_ref[...]
    e = jnp.maximum(z, 0.2 * z)
    pf = jnp.exp2(e - m_ref[...])

    delta = j * jb - i * ib

    def _update(with_diag):
        mask = a != 0
        if with_diag:
            d0 = (jax.lax.broadcasted_iota(jnp.int32, (ib, jb), 0)
                  - jax.lax.broadcasted_iota(jnp.int32, (ib, jb), 1))
            mask = mask | (d0 == delta)
        p = jnp.where(mask, pf, 0.0).astype(jnp.bfloat16)
        acc_ref[...] = acc_ref[...] + jax.lax.dot_general(
            g_ref[...], p, (((0,), (0,)), ((), ())),
            preferred_element_type=jnp.float32)

    is_diag = (delta > -jb) & (delta < ib)

    @pl.when(is_diag)
    def _diag():
        _update(True)

    @pl.when(jnp.logical_not(is_diag))
    def _offdiag():
        _update(False)

    @pl.when(i == ni - 1)
    def _fini():
        out_ref[...] = (acc_ref[:D_OUT, :] / (acc_ref[D_OUT:, :] + 1e-16)
                        + bias_ref[...])


@jax.jit
def kernel(x, adj, W1, b1, W2, att_src, att_dst, bias):
    n = x.shape[0]

    g, a_s, a_d, m = pl.pallas_call(
        _proj_kernel,
        out_shape=(
            jax.ShapeDtypeStruct((n, D_OUT + 1), jnp.bfloat16),
            jax.ShapeDtypeStruct((n, 1), jnp.float32),
            jax.ShapeDtypeStruct((n, 1), jnp.float32),
            jax.ShapeDtypeStruct((n, 1), jnp.float32),
        ),
    )(x, W1, b1.reshape(1, -1), W2,
      att_src.reshape(-1, 1), att_dst.reshape(-1, 1))

    ib, jb = 1024, 512
    ni = n // ib
    nj = n // jb

    out_t = pl.pallas_call(
        functools.partial(_attn_kernel, ib=ib, jb=jb, ni=ni),
        grid=(nj, ni),
        in_specs=[
            pl.BlockSpec((ib, jb), lambda j, i: (i, j)),
            pl.BlockSpec((ib, D_OUT + 1), lambda j, i: (i, 0)),
            pl.BlockSpec((ib, 1), lambda j, i: (i, 0)),
            pl.BlockSpec((1, jb), lambda j, i: (0, j)),
            pl.BlockSpec((1, jb), lambda j, i: (0, j)),
            pl.BlockSpec((D_OUT, 1), lambda j, i: (0, 0)),
        ],
        out_specs=pl.BlockSpec((D_OUT, jb), lambda j, i: (0, j)),
        out_shape=jax.ShapeDtypeStruct((D_OUT, n), jnp.float32),
        scratch_shapes=[
            pltpu.VMEM((D_OUT + 1, jb), jnp.float32),
        ],
        compiler_params=pltpu.CompilerParams(
            dimension_semantics=("parallel", "arbitrary")),
    )(adj, g, a_s, a_d.reshape(1, -1), m.reshape(1, -1), bias.reshape(-1, 1))

    return out_t.T


# speedup vs baseline: 1.2031x; 1.0583x over previous
import functools

import jax
import jax.numpy as jnp
from jax.experimental import pallas as pl
from jax.experimental.pallas import tpu as pltpu

D_OUT = 8
LOG2E = 1.4426950408889634


def _proj_kernel(x_ref, w1_ref, b1_ref, w2_ref, asrc_ref, adst_ref,
                 gt_ref, a_s_ref, a_d_ref, m_ref):
    ht = jax.lax.dot_general(w1_ref[...], x_ref[...], (((1,), (1,)), ((), ())),
                             preferred_element_type=jnp.float32)
    ht = ht + b1_ref[...]
    ht = jnp.where(ht >= 0, ht, 0.01 * ht)
    gt = jax.lax.dot_general(w2_ref[...], ht, (((1,), (0,)), ((), ())),
                             preferred_element_type=jnp.float32)
    gt_ref[:D_OUT, :] = gt.astype(jnp.bfloat16)
    gt_ref[D_OUT:, :] = jnp.ones_like(gt_ref[D_OUT:, :])
    g = jax.lax.dot_general(ht, w2_ref[...], (((0,), (1,)), ((), ())),
                            preferred_element_type=jnp.float32)
    a_s = LOG2E * jax.lax.dot_general(g, asrc_ref[...], (((1,), (0,)), ((), ())),
                                      preferred_element_type=jnp.float32)
    a_s_ref[...] = a_s
    a_d = LOG2E * jax.lax.dot_general(adst_ref[...], gt, (((1,), (0,)), ((), ())),
                                      preferred_element_type=jnp.float32)
    a_d_ref[...] = a_d
    t = jnp.max(a_s) + a_d
    m_ref[...] = jnp.maximum(t, 0.2 * t)


def _attn_kernel(adj_ref, gt_ref, a_s_ref, a_d_ref, m_ref, bias_ref, out_ref,
                 acc_ref, *, ib, jb, ni):
    j = pl.program_id(0)
    i = pl.program_id(1)

    @pl.when(i == 0)
    def _init():
        acc_ref[...] = jnp.zeros_like(acc_ref)

    a = adj_ref[...]
    z = a_s_ref[...] + a_d_ref[...]
    e = jnp.maximum(z, 0.2 * z)
    pf = jnp.exp2(e - m_ref[...])

    d0 = (jax.lax.broadcasted_iota(jnp.int32, (ib, jb), 0)
          - jax.lax.broadcasted_iota(jnp.int32, (ib, jb), 1))
    mask = (a != 0) | (d0 == (j * jb - i * ib))
    p = jnp.where(mask, pf, 0.0).astype(jnp.bfloat16)

    acc_ref[...] = acc_ref[...] + jax.lax.dot_general(
        gt_ref[...], p, (((1,), (0,)), ((), ())),
        preferred_element_type=jnp.float32)

    @pl.when(i == ni - 1)
    def _fini():
        o = (acc_ref[:D_OUT, :] / (acc_ref[D_OUT:, :] + 1e-16)
             + bias_ref[...])
        out_ref[...] = o.T


@jax.jit
def kernel(x, adj, W1, b1, W2, att_src, att_dst, bias):
    n = x.shape[0]

    gt, a_s, a_d, m = pl.pallas_call(
        _proj_kernel,
        out_shape=(
            jax.ShapeDtypeStruct((D_OUT + 1, n), jnp.bfloat16),
            jax.ShapeDtypeStruct((n, 1), jnp.float32),
            jax.ShapeDtypeStruct((1, n), jnp.float32),
            jax.ShapeDtypeStruct((1, n), jnp.float32),
        ),
    )(x, W1, b1.reshape(-1, 1), W2,
      att_src.reshape(-1, 1), att_dst.reshape(1, -1))

    ib, jb = 1024, 512
    ni = n // ib
    nj = n // jb

    out = pl.pallas_call(
        functools.partial(_attn_kernel, ib=ib, jb=jb, ni=ni),
        grid=(nj, ni),
        in_specs=[
            pl.BlockSpec((ib, jb), lambda j, i: (i, j)),
            pl.BlockSpec((D_OUT + 1, ib), lambda j, i: (0, i)),
            pl.BlockSpec((ib, 1), lambda j, i: (i, 0)),
            pl.BlockSpec((1, jb), lambda j, i: (0, j)),
            pl.BlockSpec((1, jb), lambda j, i: (0, j)),
            pl.BlockSpec((D_OUT, 1), lambda j, i: (0, 0)),
        ],
        out_specs=pl.BlockSpec((jb, D_OUT), lambda j, i: (j, 0)),
        out_shape=jax.ShapeDtypeStruct((n, D_OUT), jnp.float32),
        scratch_shapes=[
            pltpu.VMEM((D_OUT + 1, jb), jnp.float32),
        ],
        compiler_params=pltpu.CompilerParams(
            dimension_semantics=("parallel", "arbitrary")),
    )(adj, gt, a_s, a_d, m, bias.reshape(-1, 1))

    return out


# speedup vs baseline: 1.6802x; 1.3966x over previous
import functools

import jax
import jax.numpy as jnp
from jax.experimental import pallas as pl
from jax.experimental.pallas import tpu as pltpu

D_OUT = 8
LOG2E = 1.4426950408889634


def _proj_kernel(x_ref, w1_ref, b1_ref, w2_ref, asrc_ref, adst_ref,
                 gt_ref, gtf_ref, a_s_ref, a_sr_ref, a_d_ref, m_ref):
    ht = jax.lax.dot_general(w1_ref[...], x_ref[...], (((1,), (1,)), ((), ())),
                             preferred_element_type=jnp.float32)
    ht = ht + b1_ref[...]
    ht = jnp.where(ht >= 0, ht, 0.01 * ht)
    gt = jax.lax.dot_general(w2_ref[...], ht, (((1,), (0,)), ((), ())),
                             preferred_element_type=jnp.float32)
    gt_ref[:D_OUT, :] = gt.astype(jnp.bfloat16)
    gt_ref[D_OUT:, :] = jnp.ones_like(gt_ref[D_OUT:, :])
    gtf_ref[...] = gt
    g = jax.lax.dot_general(ht, w2_ref[...], (((0,), (1,)), ((), ())),
                            preferred_element_type=jnp.float32)
    a_s = LOG2E * jax.lax.dot_general(g, asrc_ref[...], (((1,), (1,)), ((), ())),
                                      preferred_element_type=jnp.float32)
    a_s_ref[...] = a_s
    a_sr = LOG2E * jax.lax.dot_general(asrc_ref[...], gt, (((1,), (0,)), ((), ())),
                                       preferred_element_type=jnp.float32)
    a_sr_ref[...] = a_sr
    a_d = LOG2E * jax.lax.dot_general(adst_ref[...], gt, (((1,), (0,)), ((), ())),
                                      preferred_element_type=jnp.float32)
    a_d_ref[...] = a_d
    t = jnp.max(a_s) + a_d
    m_ref[...] = jnp.maximum(t, 0.2 * t)


def _attn_kernel(adj_ref, gt_ref, gtf_ref, a_s_ref, a_sr_ref, a_d_ref, m_ref,
                 bias_ref, out_ref, acc_ref, diag_ref, *, ib, ni):
    i = pl.program_id(0)

    @pl.when(i == 0)
    def _init():
        acc_ref[...] = jnp.zeros_like(acc_ref)

    a = adj_ref[...]
    z = a_s_ref[...] + a_d_ref[...]
    e = jnp.maximum(z, 0.2 * z)
    pf = jnp.exp2(e - m_ref[...])
    p = jnp.where(a != 0, pf, 0.0).astype(jnp.bfloat16)

    acc_ref[...] = acc_ref[...] + jax.lax.dot_general(
        gt_ref[...], p, (((1,), (0,)), ((), ())),
        preferred_element_type=jnp.float32)

    asub = adj_ref[:, pl.ds(i * ib, ib)]
    d0 = (jax.lax.broadcasted_iota(jnp.int32, (ib, ib), 0)
          - jax.lax.broadcasted_iota(jnp.int32, (ib, ib), 1))
    diag_ref[:, pl.ds(i * ib, ib)] = jnp.sum(
        jnp.where(d0 == 0, asub, 0), axis=0, keepdims=True)

    @pl.when(i == ni - 1)
    def _fini():
        t = a_sr_ref[...] + a_d_ref[...]
        ed = jnp.maximum(t, 0.2 * t)
        pfd = jnp.exp2(ed - m_ref[...])
        w = jnp.where(diag_ref[...] != 0, 0.0, pfd)
        s = acc_ref[D_OUT:, :] + w
        num = acc_ref[:D_OUT, :] + gtf_ref[...] * w
        o = num / (s + 1e-16) + bias_ref[...]
        out_ref[...] = o.T


@jax.jit
def kernel(x, adj, W1, b1, W2, att_src, att_dst, bias):
    n = x.shape[0]

    gt, gtf, a_s, a_sr, a_d, m = pl.pallas_call(
        _proj_kernel,
        out_shape=(
            jax.ShapeDtypeStruct((D_OUT + 1, n), jnp.bfloat16),
            jax.ShapeDtypeStruct((D_OUT, n), jnp.float32),
            jax.ShapeDtypeStruct((n, 1), jnp.float32),
            jax.ShapeDtypeStruct((1, n), jnp.float32),
            jax.ShapeDtypeStruct((1, n), jnp.float32),
            jax.ShapeDtypeStruct((1, n), jnp.float32),
        ),
    )(x, W1, b1.reshape(-1, 1), W2,
      att_src.reshape(1, -1), att_dst.reshape(1, -1))

    ib = 256
    ni = n // ib

    out = pl.pallas_call(
        functools.partial(_attn_kernel, ib=ib, ni=ni),
        grid=(ni,),
        in_specs=[
            pl.BlockSpec((ib, n), lambda i: (i, 0)),
            pl.BlockSpec((D_OUT + 1, ib), lambda i: (0, i)),
            pl.BlockSpec((D_OUT, n), lambda i: (0, 0)),
            pl.BlockSpec((ib, 1), lambda i: (i, 0)),
            pl.BlockSpec((1, n), lambda i: (0, 0)),
            pl.BlockSpec((1, n), lambda i: (0, 0)),
            pl.BlockSpec((1, n), lambda i: (0, 0)),
            pl.BlockSpec((D_OUT, 1), lambda i: (0, 0)),
        ],
        out_specs=pl.BlockSpec((n, D_OUT), lambda i: (0, 0)),
        out_shape=jax.ShapeDtypeStruct((n, D_OUT), jnp.float32),
        scratch_shapes=[
            pltpu.VMEM((D_OUT + 1, n), jnp.float32),
            pltpu.VMEM((1, n), jnp.int32),
        ],
    )(adj, gt, gtf, a_s, a_sr, a_d, m, bias.reshape(-1, 1))

    return out


# speedup vs baseline: 1.8578x; 1.1057x over previous
import functools

import jax
import jax.numpy as jnp
from jax.experimental import pallas as pl
from jax.experimental.pallas import tpu as pltpu

D_OUT = 8
LOG2E = 1.4426950408889634


def _proj_kernel(x_ref, w1_ref, b1_ref, w2_ref, asrc_ref, adst_ref,
                 gt_ref, gtf_ref, a_s_ref, a_sr_ref, ad2_ref, ad3_ref):
    ht = jax.lax.dot_general(w1_ref[...], x_ref[...], (((1,), (1,)), ((), ())),
                             preferred_element_type=jnp.float32)
    ht = ht + b1_ref[...]
    ht = jnp.where(ht >= 0, ht, 0.01 * ht)
    gt = jax.lax.dot_general(w2_ref[...], ht, (((1,), (0,)), ((), ())),
                             preferred_element_type=jnp.float32)
    gt_ref[:D_OUT, :] = gt.astype(jnp.bfloat16)
    gt_ref[D_OUT:, :] = jnp.ones_like(gt_ref[D_OUT:, :])
    gtf_ref[...] = gt
    g = jax.lax.dot_general(ht, w2_ref[...], (((0,), (1,)), ((), ())),
                            preferred_element_type=jnp.float32)
    a_s = LOG2E * jax.lax.dot_general(g, asrc_ref[...], (((1,), (1,)), ((), ())),
                                      preferred_element_type=jnp.float32)
    a_s_ref[...] = a_s
    a_sr = LOG2E * jax.lax.dot_general(asrc_ref[...], gt, (((1,), (0,)), ((), ())),
                                       preferred_element_type=jnp.float32)
    a_sr_ref[...] = a_sr
    a_d = LOG2E * jax.lax.dot_general(adst_ref[...], gt, (((1,), (0,)), ((), ())),
                                      preferred_element_type=jnp.float32)
    t = jnp.max(a_s) + a_d
    m = jnp.maximum(t, 0.2 * t)
    ad2_ref[...] = a_d - m
    ad3_ref[...] = 0.2 * a_d - m


def _attn_kernel(adj0_ref, adj1_ref, gt0_ref, gt1_ref, a_s0_ref, a_s1_ref,
                 a_sr_ref, ad2_ref, ad3_ref, gtf_ref, bias_ref, out_ref,
                 acc_ref, diag_ref, *, ib, ni):
    i = pl.program_id(0)

    @pl.when(i == 0)
    def _init():
        acc_ref[...] = jnp.zeros_like(acc_ref)

    ad2 = ad2_ref[...]
    ad3 = ad3_ref[...]
    d0 = (jax.lax.broadcasted_iota(jnp.int32, (ib, ib), 0)
          - jax.lax.broadcasted_iota(jnp.int32, (ib, ib), 1))

    def _strip(adj_ref, gt_ref, a_s_ref, base):
        a = adj_ref[...]
        asv = a_s_ref[...]
        u = jnp.maximum(asv + ad2, 0.2 * asv + ad3)
        pf = jnp.exp2(u)
        p = jnp.where(a != 0, pf, 0.0).astype(jnp.bfloat16)
        acc_ref[...] = acc_ref[...] + jax.lax.dot_general(
            gt_ref[...], p, (((1,), (0,)), ((), ())),
            preferred_element_type=jnp.float32)
        asub = adj_ref[:, pl.ds(base * ib, ib)]
        diag_ref[:, pl.ds(base * ib, ib)] = jnp.sum(
            jnp.where(d0 == 0, asub, 0), axis=0, keepdims=True)

    _strip(adj0_ref, gt0_ref, a_s0_ref, i)
    _strip(adj1_ref, gt1_ref, a_s1_ref, i + ni)

    @pl.when(i == ni - 1)
    def _fini():
        asr = a_sr_ref[...]
        ud = jnp.maximum(asr + ad2, 0.2 * asr + ad3)
        pfd = jnp.exp2(ud)
        w = jnp.where(diag_ref[...] != 0, 0.0, pfd)
        s = acc_ref[D_OUT:, :] + w
        num = acc_ref[:D_OUT, :] + gtf_ref[...] * w
        o = num / (s + 1e-16) + bias_ref[...]
        out_ref[...] = o.T


@jax.jit
def kernel(x, adj, W1, b1, W2, att_src, att_dst, bias):
    n = x.shape[0]

    gt, gtf, a_s, a_sr, ad2, ad3 = pl.pallas_call(
        _proj_kernel,
        out_shape=(
            jax.ShapeDtypeStruct((D_OUT + 1, n), jnp.bfloat16),
            jax.ShapeDtypeStruct((D_OUT, n), jnp.float32),
            jax.ShapeDtypeStruct((n, 1), jnp.float32),
            jax.ShapeDtypeStruct((1, n), jnp.float32),
            jax.ShapeDtypeStruct((1, n), jnp.float32),
            jax.ShapeDtypeStruct((1, n), jnp.float32),
        ),
    )(x, W1, b1.reshape(-1, 1), W2,
      att_src.reshape(1, -1), att_dst.reshape(1, -1))

    ib = 256
    ni = n // ib // 2

    out = pl.pallas_call(
        functools.partial(_attn_kernel, ib=ib, ni=ni),
        grid=(ni,),
        in_specs=[
            pl.BlockSpec((ib, n), lambda i: (i, 0)),
            pl.BlockSpec((ib, n), lambda i: (i + 8, 0)),
            pl.BlockSpec((D_OUT + 1, ib), lambda i: (0, i)),
            pl.BlockSpec((D_OUT + 1, ib), lambda i: (0, i + 8)),
            pl.BlockSpec((ib, 1), lambda i: (i, 0)),
            pl.BlockSpec((ib, 1), lambda i: (i + 8, 0)),
            pl.BlockSpec((1, n), lambda i: (0, 0)),
            pl.BlockSpec((1, n), lambda i: (0, 0)),
            pl.BlockSpec((1, n), lambda i: (0, 0)),
            pl.BlockSpec((D_OUT, n), lambda i: (0, 0)),
            pl.BlockSpec((D_OUT, 1), lambda i: (0, 0)),
        ],
        out_specs=pl.BlockSpec((n, D_OUT), lambda i: (0, 0)),
        out_shape=jax.ShapeDtypeStruct((n, D_OUT), jnp.float32),
        scratch_shapes=[
            pltpu.VMEM((D_OUT + 1, n), jnp.float32),
            pltpu.VMEM((1, n), jnp.int32),
        ],
    )(adj, adj, gt, gt, a_s, a_s, a_sr, ad2, ad3, gtf, bias.reshape(-1, 1))

    return out
